# baseline (speedup 1.0000x reference)
import functools

import jax
import jax.numpy as jnp
from jax import lax
from jax.experimental import pallas as pl
from jax.experimental.pallas import tpu as pltpu
from jax.experimental.pallas import tpu_sc as plsc

N = 10000
E = 320000
D = 128
HD = D // 2
L = 16
CH = 128
EPT = E // 16
SEGE = 4992
SEG = SEGE // CH
TAIL = EPT - 4 * SEGE
EPC = EPT // 2
DCH = EPC // CH
NP = 10112
RPT = NP // 16
YPT = N // 16
YC = YPT // 5
TCB = 1000


def _deg_body(eix_hbm, ones_hbm, zeros_hbm, deg_out, deg_sh, dstbuf, ones_v):
    c = lax.axis_index("c")
    s = lax.axis_index("s")
    pltpu.sync_copy(ones_hbm, ones_v)
    pltpu.sync_copy(zeros_hbm, deg_sh.at[pl.ds(s * RPT, RPT)])
    plsc.subcore_barrier()

    pltpu.sync_copy(eix_hbm.at[1, pl.ds(s * EPT + c * EPC, EPC)], dstbuf)

    def body(k, _):
        pltpu.sync_copy(ones_v, deg_sh.at[dstbuf.at[pl.ds(k * CH, CH)]],
                        add=True)
        return 0

    lax.fori_loop(0, DCH, body, 0)
    pltpu.sync_copy(ones_v.at[pl.ds(0, EPC - DCH * CH)],
                    deg_sh.at[dstbuf.at[pl.ds(DCH * CH, EPC - DCH * CH)]],
                    add=True)
    plsc.subcore_barrier()
    pltpu.sync_copy(deg_sh.at[pl.ds(s * RPT, RPT)],
                    deg_out.at[c, pl.ds(s * RPT, RPT)])


def _edge_body(eix_hbm, y2_hbm, base2_hbm, dis_hbm, zrows_hbm, out_hbm,
               z_sh, y_sh, srcbuf, dstbuf, rb0, rb1, rb2,
               g0, g1, g2, t0, t1, t2):
    c = lax.axis_index("c")
    s = lax.axis_index("s")
    rbufs = (rb0, rb1, rb2)
    gsems = (g0, g1, g2)
    ssems = (t0, t1, t2)

    zslice = rb0.at[pl.ds(0, RPT // 8)]
    pltpu.sync_copy(zrows_hbm, zslice)
    for t in range(8):
        pltpu.sync_copy(zslice, z_sh.at[pl.ds(s * RPT + t * (RPT // 8), RPT // 8)])

    ybounce = rb1.at[pl.ds(0, YC)]
    for t in range(5):
        off = s * YPT + t * YC
        pltpu.sync_copy(y2_hbm.at[c, pl.ds(off, YC)], ybounce)
        pltpu.sync_copy(ybounce, y_sh.at[pl.ds(off, YC)])

    plsc.subcore_barrier()

    for h in range(4):
        base_e = s * EPT + h * SEGE
        pltpu.sync_copy(eix_hbm.at[0, pl.ds(base_e, SEGE)],
                        srcbuf.at[pl.ds(0, SEGE)])
        pltpu.sync_copy(eix_hbm.at[1, pl.ds(base_e, SEGE)],
                        dstbuf.at[pl.ds(0, SEGE)])
        if h == 3:
            pltpu.sync_copy(eix_hbm.at[0, pl.ds(base_e + SEGE, TAIL)],
                            srcbuf.at[pl.ds(SEGE, TAIL)])
            pltpu.sync_copy(eix_hbm.at[1, pl.ds(base_e + SEGE, TAIL)],
                            dstbuf.at[pl.ds(SEGE, TAIL)])

        pltpu.async_copy(y_sh.at[srcbuf.at[pl.ds(0, CH)]], rbufs[0], gsems[0])
        pltpu.async_copy(y_sh.at[srcbuf.at[pl.ds(CH, CH)]], rbufs[1], gsems[1])

        def body(i, _):
            for d in range(3):
                k = 3 * i + d
                b = d
                bn = (d + 2) % 3
                if d == 0:
                    @pl.when(i > 0)
                    def _():
                        pltpu.make_async_copy(
                            rbufs[bn],
                            z_sh.at[dstbuf.at[pl.ds((k - 1) * CH, CH)]],
                            ssems[bn]).wait()

                    pltpu.async_copy(y_sh.at[srcbuf.at[pl.ds((k + 2) * CH, CH)]],
                                     rbufs[bn], gsems[bn])
                else:
                    pltpu.make_async_copy(
                        rbufs[bn],
                        z_sh.at[dstbuf.at[pl.ds((k - 1) * CH, CH)]],
                        ssems[bn]).wait()

                    @pl.when(i < SEG // 3 - 1)
                    def _():
                        pltpu.async_copy(
                            y_sh.at[srcbuf.at[pl.ds((k + 2) * CH, CH)]],
                            rbufs[bn], gsems[bn])

                pltpu.make_async_copy(y_sh.at[srcbuf.at[pl.ds(k * CH, CH)]],
                                      rbufs[b], gsems[b]).wait()
                pltpu.async_copy(rbufs[b],
                                 z_sh.at[dstbuf.at[pl.ds(k * CH, CH)]],
                                 ssems[b], add=True)
            return 0

        lax.fori_loop(0, SEG // 3, body, 0)
        pltpu.make_async_copy(
            rbufs[(SEG - 1) % 3],
            z_sh.at[dstbuf.at[pl.ds((SEG - 1) * CH, CH)]],
            ssems[(SEG - 1) % 3]).wait()
        if h == 3:
            tb = rbufs[0].at[pl.ds(0, TAIL)]
            pltpu.async_copy(y_sh.at[srcbuf.at[pl.ds(SEGE, TAIL)]],
                             tb, gsems[0]).wait()
            pltpu.async_copy(tb, z_sh.at[dstbuf.at[pl.ds(SEGE, TAIL)]],
                             ssems[0], add=True).wait()

    plsc.subcore_barrier()

    bz = rb0.at[pl.ds(0, YC)]
    bb = rb1.at[pl.ds(0, YC)]
    bd = rb2.at[pl.ds(0, YC), pl.ds(0, L)]
    for t in range(5):
        off = s * YPT + t * YC
        pltpu.sync_copy(z_sh.at[pl.ds(off, YC)], bz)
        pltpu.sync_copy(base2_hbm.at[c, pl.ds(off, YC)], bb)
        pltpu.sync_copy(dis_hbm.at[pl.ds(off, YC)], bd)

        def row(r, _):
            dv = bd[r, :]
            for j in range(HD // L):
                sl = pl.ds(j * L, L)
                bz[r, sl] = bz[r, sl] * dv + bb[r, sl]
            return 0

        lax.fori_loop(0, YC, row, 0)
        pltpu.sync_copy(bz, out_hbm.at[c, pl.ds(off, YC)])


@functools.cache
def _build_sc_kernels():
    mesh = plsc.VectorSubcoreMesh(core_axis_name="c", subcore_axis_name="s",
                                  num_cores=2, num_subcores=16)
    params = pltpu.CompilerParams(use_tc_tiling_on_sc=False)
    deg_kernel = pl.kernel(
        _deg_body,
        out_type=jax.ShapeDtypeStruct((2, NP, L), jnp.float32),
        mesh=mesh,
        compiler_params=params,
        scratch_types=[
            pltpu.VMEM_SHARED((NP, L), jnp.float32),
            pltpu.VMEM((EPC,), jnp.int32),
            pltpu.VMEM((CH, L), jnp.float32),
        ],
    )
    edge_kernel = pl.kernel(
        _edge_body,
        out_type=jax.ShapeDtypeStruct((2, N, HD), jnp.float32),
        mesh=mesh,
        compiler_params=params,
        scratch_types=[
            pltpu.VMEM_SHARED((NP, HD), jnp.float32),
            pltpu.VMEM_SHARED((N, HD), jnp.float32),
            pltpu.VMEM((SEGE + TAIL,), jnp.int32),
            pltpu.VMEM((SEGE + TAIL,), jnp.int32),
            pltpu.VMEM((CH, HD), jnp.float32),
            pltpu.VMEM((CH, HD), jnp.float32),
            pltpu.VMEM((CH, HD), jnp.float32),
            pltpu.SemaphoreType.DMA,
            pltpu.SemaphoreType.DMA,
            pltpu.SemaphoreType.DMA,
            pltpu.SemaphoreType.DMA,
            pltpu.SemaphoreType.DMA,
            pltpu.SemaphoreType.DMA,
        ],
    )
    return deg_kernel, edge_kernel


def _tc_prep_body(x_ref, h_ref, w_ref, d0_ref, d1_ref, b_ref,
                  y2_ref, base2_ref, dis_ref):
    comb = x_ref[...] + h_ref[...]
    xw = lax.dot_general(comb, w_ref[...], (((1,), (0,)), ((), ())),
                         precision=lax.Precision.HIGHEST,
                         preferred_element_type=jnp.float32)
    deg = d0_ref[0, :, 0:1] + d1_ref[0, :, 0:1] + 1.0
    dis = lax.rsqrt(deg)
    y = xw * dis
    base = xw * (dis * dis) + b_ref[...]
    y2_ref[0] = y[:, :HD]
    y2_ref[1] = y[:, HD:]
    base2_ref[0] = base[:, :HD]
    base2_ref[1] = base[:, HD:]
    dis_ref[...] = jnp.broadcast_to(dis, (TCB, L))


def kernel(x, edge_index, h_cur, c_cur, W, b):
    eix = edge_index.astype(jnp.int32)
    W128 = W[:, :D]
    b128 = b[:D].reshape(1, D)

    deg_kernel, edge_kernel = _build_sc_kernels()
    ones16 = jnp.ones((CH, L), jnp.float32)
    zeros16 = jnp.zeros((RPT, L), jnp.float32)
    zrows = jnp.zeros((RPT // 8, HD), jnp.float32)
    deg2 = deg_kernel(eix, ones16, zeros16)

    row_spec = pl.BlockSpec((TCB, D), lambda i: (i, 0))
    half2_spec = pl.BlockSpec((2, TCB, HD), lambda i: (0, i, 0))
    deg0_spec = pl.BlockSpec((1, TCB, L), lambda i: (0, i, 0))
    deg1_spec = pl.BlockSpec((1, TCB, L), lambda i: (1, i, 0))
    y2, base2, dis16 = pl.pallas_call(
        _tc_prep_body,
        grid=(N // TCB,),
        in_specs=[
            row_spec,
            row_spec,
            pl.BlockSpec((D, D), lambda i: (0, 0)),
            deg0_spec,
            deg1_spec,
            pl.BlockSpec((1, D), lambda i: (0, 0)),
        ],
        out_specs=[half2_spec, half2_spec,
                   pl.BlockSpec((TCB, L), lambda i: (i, 0))],
        out_shape=[jax.ShapeDtypeStruct((2, N, HD), jnp.float32),
                   jax.ShapeDtypeStruct((2, N, HD), jnp.float32),
                   jax.ShapeDtypeStruct((N, L), jnp.float32)],
    )(x, h_cur, W128, deg2, deg2, b128)

    o2 = edge_kernel(eix, y2, base2, dis16, zrows)
    return jnp.concatenate([o2[0], o2[1]], axis=1)


# speedup vs baseline: 1.1249x; 1.1249x over previous
import functools

import jax
import jax.numpy as jnp
from jax import lax
from jax.experimental import pallas as pl
from jax.experimental.pallas import tpu as pltpu
from jax.experimental.pallas import tpu_sc as plsc

N = 10000
E = 320000
D = 128
HD = D // 2
L = 16
CH = 128
EPT = E // 16
SEG = 78
SEGE = SEG * CH
TAIL = EPT - 2 * SEGE
EPC = EPT // 2
DCH = EPC // CH
NP = 10112
RPT = NP // 16
YPT = N // 16
YC = YPT // 5
TCB = 1000


def _deg_body(eix_hbm, ones_hbm, zeros_hbm, deg_out, deg_sh, dstbuf, ones_v):
    c = lax.axis_index("c")
    s = lax.axis_index("s")
    pltpu.sync_copy(ones_hbm, ones_v)
    pltpu.sync_copy(zeros_hbm, deg_sh.at[pl.ds(s * RPT, RPT)])
    plsc.subcore_barrier()

    pltpu.sync_copy(eix_hbm.at[1, pl.ds(s * EPT + c * EPC, EPC)], dstbuf)

    def body(k, _):
        pltpu.sync_copy(ones_v, deg_sh.at[dstbuf.at[pl.ds(k * CH, CH)]],
                        add=True)
        return 0

    lax.fori_loop(0, DCH, body, 0)
    pltpu.sync_copy(ones_v.at[pl.ds(0, EPC - DCH * CH)],
                    deg_sh.at[dstbuf.at[pl.ds(DCH * CH, EPC - DCH * CH)]],
                    add=True)
    plsc.subcore_barrier()
    pltpu.sync_copy(deg_sh.at[pl.ds(s * RPT, RPT)],
                    deg_out.at[c, pl.ds(s * RPT, RPT)])


def _edge_body(eix_hbm, y2_hbm, zrows_hbm, z_out,
               z_sh, y_sh, srcbuf, dstbuf, rb0, rb1, rb2,
               g0, g1, g2, t0, t1, t2):
    c = lax.axis_index("c")
    s = lax.axis_index("s")
    rbufs = (rb0, rb1, rb2)
    gsems = (g0, g1, g2)
    ssems = (t0, t1, t2)

    zb = rb0.at[pl.ds(0, YC)]
    pltpu.sync_copy(zrows_hbm, zb)
    for t in range(5):
        pltpu.async_copy(zb, z_sh.at[pl.ds(s * YPT + t * YC, YC)], g0)

    yb = rb1.at[pl.ds(0, YC)]
    for t in range(5):
        off = s * YPT + t * YC
        pltpu.sync_copy(y2_hbm.at[c, pl.ds(off, YC)], yb)
        pltpu.sync_copy(yb, y_sh.at[pl.ds(off, YC)])
    for t in range(5):
        pltpu.make_async_copy(zb, z_sh.at[pl.ds(s * YPT + t * YC, YC)],
                              g0).wait()

    plsc.subcore_barrier()

    for h in range(2):
        base_e = s * EPT + h * SEGE
        pltpu.sync_copy(eix_hbm.at[0, pl.ds(base_e, SEGE)],
                        srcbuf.at[pl.ds(0, SEGE)])
        pltpu.sync_copy(eix_hbm.at[1, pl.ds(base_e, SEGE)],
                        dstbuf.at[pl.ds(0, SEGE)])
        if h == 1:
            pltpu.sync_copy(eix_hbm.at[0, pl.ds(base_e + SEGE, TAIL)],
                            srcbuf.at[pl.ds(SEGE, TAIL)])
            pltpu.sync_copy(eix_hbm.at[1, pl.ds(base_e + SEGE, TAIL)],
                            dstbuf.at[pl.ds(SEGE, TAIL)])

        pltpu.async_copy(y_sh.at[srcbuf.at[pl.ds(0, CH)]], rbufs[0], gsems[0])
        pltpu.async_copy(y_sh.at[srcbuf.at[pl.ds(CH, CH)]], rbufs[1], gsems[1])

        def body(i, _):
            for d in range(3):
                k = 3 * i + d
                b = d
                bn = (d + 2) % 3
                if d == 0:
                    @pl.when(i > 0)
                    def _():
                        pltpu.make_async_copy(
                            rbufs[bn],
                            z_sh.at[dstbuf.at[pl.ds((k - 1) * CH, CH)]],
                            ssems[bn]).wait()

                    pltpu.async_copy(y_sh.at[srcbuf.at[pl.ds((k + 2) * CH, CH)]],
                                     rbufs[bn], gsems[bn])
                else:
                    pltpu.make_async_copy(
                        rbufs[bn],
                        z_sh.at[dstbuf.at[pl.ds((k - 1) * CH, CH)]],
                        ssems[bn]).wait()

                    @pl.when(i < SEG // 3 - 1)
                    def _():
                        pltpu.async_copy(
                            y_sh.at[srcbuf.at[pl.ds((k + 2) * CH, CH)]],
                            rbufs[bn], gsems[bn])

                pltpu.make_async_copy(y_sh.at[srcbuf.at[pl.ds(k * CH, CH)]],
                                      rbufs[b], gsems[b]).wait()
                pltpu.async_copy(rbufs[b],
                                 z_sh.at[dstbuf.at[pl.ds(k * CH, CH)]],
                                 ssems[b], add=True)
            return 0

        lax.fori_loop(0, SEG // 3, body, 0)
        pltpu.make_async_copy(
            rbufs[(SEG - 1) % 3],
            z_sh.at[dstbuf.at[pl.ds((SEG - 1) * CH, CH)]],
            ssems[(SEG - 1) % 3]).wait()
        if h == 1:
            tb = rbufs[0].at[pl.ds(0, TAIL)]
            pltpu.async_copy(y_sh.at[srcbuf.at[pl.ds(SEGE, TAIL)]],
                             tb, gsems[0]).wait()
            pltpu.async_copy(tb, z_sh.at[dstbuf.at[pl.ds(SEGE, TAIL)]],
                             ssems[0], add=True).wait()

    plsc.subcore_barrier()
    for t in range(5):
        off = s * YPT + t * YC
        pltpu.sync_copy(z_sh.at[pl.ds(off, YC)], zb)
        pltpu.sync_copy(zb, z_out.at[c, pl.ds(off, YC)])


@functools.cache
def _build_sc_kernels():
    mesh = plsc.VectorSubcoreMesh(core_axis_name="c", subcore_axis_name="s",
                                  num_cores=2, num_subcores=16)
    params = pltpu.CompilerParams(use_tc_tiling_on_sc=False)
    deg_kernel = pl.kernel(
        _deg_body,
        out_type=jax.ShapeDtypeStruct((2, NP, L), jnp.float32),
        mesh=mesh,
        compiler_params=params,
        scratch_types=[
            pltpu.VMEM_SHARED((NP, L), jnp.float32),
            pltpu.VMEM((EPC,), jnp.int32),
            pltpu.VMEM((CH, L), jnp.float32),
        ],
    )
    edge_kernel = pl.kernel(
        _edge_body,
        out_type=jax.ShapeDtypeStruct((2, N, HD), jnp.float32),
        mesh=mesh,
        compiler_params=params,
        scratch_types=[
            pltpu.VMEM_SHARED((N, HD), jnp.float32),
            pltpu.VMEM_SHARED((N, HD), jnp.float32),
            pltpu.VMEM((SEGE + TAIL,), jnp.int32),
            pltpu.VMEM((SEGE + TAIL,), jnp.int32),
            pltpu.VMEM((CH, HD), jnp.float32),
            pltpu.VMEM((CH, HD), jnp.float32),
            pltpu.VMEM((CH, HD), jnp.float32),
            pltpu.SemaphoreType.DMA,
            pltpu.SemaphoreType.DMA,
            pltpu.SemaphoreType.DMA,
            pltpu.SemaphoreType.DMA,
            pltpu.SemaphoreType.DMA,
            pltpu.SemaphoreType.DMA,
        ],
    )
    return deg_kernel, edge_kernel


def _tc_prep_body(x_ref, h_ref, w_ref, d0_ref, d1_ref, b_ref,
                  y2_ref, base_ref):
    comb = x_ref[...] + h_ref[...]
    xw = lax.dot_general(comb, w_ref[...], (((1,), (0,)), ((), ())),
                         precision=lax.Precision.HIGHEST,
                         preferred_element_type=jnp.float32)
    deg = d0_ref[0, :, 0:1] + d1_ref[0, :, 0:1] + 1.0
    dis = lax.rsqrt(deg)
    y = xw * dis
    y2_ref[0] = y[:, :HD]
    y2_ref[1] = y[:, HD:]
    base_ref[...] = xw * (dis * dis) + b_ref[...]


def _tc_fin_body(z0_ref, z1_ref, d0_ref, d1_ref, base_ref, o_ref):
    deg = d0_ref[0, :, 0:1] + d1_ref[0, :, 0:1] + 1.0
    dis = lax.rsqrt(deg)
    z = jnp.concatenate([z0_ref[0], z1_ref[0]], axis=1)
    o_ref[...] = z * dis + base_ref[...]


def kernel(x, edge_index, h_cur, c_cur, W, b):
    eix = edge_index.astype(jnp.int32)
    W128 = W[:, :D]
    b128 = b[:D].reshape(1, D)

    deg_kernel, edge_kernel = _build_sc_kernels()
    ones16 = jnp.ones((CH, L), jnp.float32)
    zeros16 = jnp.zeros((RPT, L), jnp.float32)
    zrows = jnp.zeros((YC, HD), jnp.float32)
    deg2 = deg_kernel(eix, ones16, zeros16)

    row_spec = pl.BlockSpec((TCB, D), lambda i: (i, 0))
    half2_spec = pl.BlockSpec((2, TCB, HD), lambda i: (0, i, 0))
    deg0_spec = pl.BlockSpec((1, TCB, L), lambda i: (0, i, 0))
    deg1_spec = pl.BlockSpec((1, TCB, L), lambda i: (1, i, 0))
    y2, base = pl.pallas_call(
        _tc_prep_body,
        grid=(N // TCB,),
        in_specs=[
            row_spec,
            row_spec,
            pl.BlockSpec((D, D), lambda i: (0, 0)),
            deg0_spec,
            deg1_spec,
            pl.BlockSpec((1, D), lambda i: (0, 0)),
        ],
        out_specs=[half2_spec, row_spec],
        out_shape=[jax.ShapeDtypeStruct((2, N, HD), jnp.float32),
                   jax.ShapeDtypeStruct((N, D), jnp.float32)],
    )(x, h_cur, W128, deg2, deg2, b128)

    z2 = edge_kernel(eix, y2, zrows)

    z0_spec = pl.BlockSpec((1, TCB, HD), lambda i: (0, i, 0))
    z1_spec = pl.BlockSpec((1, TCB, HD), lambda i: (1, i, 0))
    out = pl.pallas_call(
        _tc_fin_body,
        grid=(N // TCB,),
        in_specs=[z0_spec, z1_spec, deg0_spec, deg1_spec, row_spec],
        out_specs=row_spec,
        out_shape=jax.ShapeDtypeStruct((N, D), jnp.float32),
    )(z2, z2, deg2, deg2, base)
    return out


# speedup vs baseline: 1.1497x; 1.0221x over previous
import functools

import jax
import jax.numpy as jnp
from jax import lax
from jax.experimental import pallas as pl
from jax.experimental.pallas import tpu as pltpu
from jax.experimental.pallas import tpu_sc as plsc

N = 10000
E = 320000
D = 128
HD = D // 2
L = 16
CH = 128
EPT = E // 16
SEG = 78
SEGE = SEG * CH
TAIL = EPT - 2 * SEGE
EPC = EPT // 2
DCH = EPC // CH
NP = 10112
RPT = NP // 16
YPT = N // 16
YC = YPT // 5
TCB = 1000


def _deg_body(eix_hbm, ones_hbm, zeros_hbm, deg_out, deg_sh, dstbuf, ones_v,
              d0s, d1s):
    c = lax.axis_index("c")
    s = lax.axis_index("s")
    pltpu.sync_copy(ones_hbm, ones_v)
    pltpu.sync_copy(zeros_hbm, deg_sh.at[pl.ds(s * RPT, RPT)])
    plsc.subcore_barrier()

    pltpu.sync_copy(eix_hbm.at[1, pl.ds(s * EPT + c * EPC, EPC)], dstbuf)
    dsems = (d0s, d1s)
    pltpu.async_copy(ones_v, deg_sh.at[dstbuf.at[pl.ds(0, CH)]], d0s,
                     add=True)
    pltpu.async_copy(ones_v, deg_sh.at[dstbuf.at[pl.ds(CH, CH)]], d1s,
                     add=True)

    def body(i, _):
        for p in range(2):
            k = 2 * i + p
            pltpu.make_async_copy(
                ones_v, deg_sh.at[dstbuf.at[pl.ds(k * CH, CH)]],
                dsems[p]).wait()
            pltpu.async_copy(
                ones_v, deg_sh.at[dstbuf.at[pl.ds((k + 2) * CH, CH)]],
                dsems[p], add=True)
        return 0

    lax.fori_loop(0, DCH // 2 - 1, body, 0)
    for p in range(2):
        k = DCH - 2 + p
        pltpu.make_async_copy(
            ones_v, deg_sh.at[dstbuf.at[pl.ds(k * CH, CH)]], dsems[p]).wait()
    pltpu.sync_copy(ones_v.at[pl.ds(0, EPC - DCH * CH)],
                    deg_sh.at[dstbuf.at[pl.ds(DCH * CH, EPC - DCH * CH)]],
                    add=True)
    plsc.subcore_barrier()
    pltpu.sync_copy(deg_sh.at[pl.ds(s * RPT, RPT)],
                    deg_out.at[c, pl.ds(s * RPT, RPT)])


def _edge_body(eix_hbm, y2_hbm, zrows_hbm, z_out,
               z_sh, y_sh, srcbuf, dstbuf, rb0, rb1, rb2,
               g0, g1, g2, t0, t1, t2):
    c = lax.axis_index("c")
    s = lax.axis_index("s")
    rbufs = (rb0, rb1, rb2)
    gsems = (g0, g1, g2)
    ssems = (t0, t1, t2)

    zb = rb0.at[pl.ds(0, YC)]
    pltpu.sync_copy(zrows_hbm, zb)
    for t in range(5):
        pltpu.async_copy(zb, z_sh.at[pl.ds(s * YPT + t * YC, YC)], g0)

    yb = rb1.at[pl.ds(0, YC)]
    for t in range(5):
        off = s * YPT + t * YC
        pltpu.sync_copy(y2_hbm.at[c, pl.ds(off, YC)], yb)
        pltpu.sync_copy(yb, y_sh.at[pl.ds(off, YC)])
    for t in range(5):
        pltpu.make_async_copy(zb, z_sh.at[pl.ds(s * YPT + t * YC, YC)],
                              g0).wait()

    plsc.subcore_barrier()

    for h in range(2):
        base_e = s * EPT + h * SEGE
        pltpu.sync_copy(eix_hbm.at[0, pl.ds(base_e, SEGE)],
                        srcbuf.at[pl.ds(0, SEGE)])
        pltpu.sync_copy(eix_hbm.at[1, pl.ds(base_e, SEGE)],
                        dstbuf.at[pl.ds(0, SEGE)])
        if h == 1:
            pltpu.sync_copy(eix_hbm.at[0, pl.ds(base_e + SEGE, TAIL)],
                            srcbuf.at[pl.ds(SEGE, TAIL)])
            pltpu.sync_copy(eix_hbm.at[1, pl.ds(base_e + SEGE, TAIL)],
                            dstbuf.at[pl.ds(SEGE, TAIL)])

        pltpu.async_copy(y_sh.at[srcbuf.at[pl.ds(0, CH)]], rbufs[0], gsems[0])
        pltpu.async_copy(y_sh.at[srcbuf.at[pl.ds(CH, CH)]], rbufs[1], gsems[1])

        def body(i, _):
            for d in range(3):
                k = 3 * i + d
                b = d
                bn = (d + 2) % 3
                if d == 0:
                    @pl.when(i > 0)
                    def _():
                        pltpu.make_async_copy(
                            rbufs[bn],
                            z_sh.at[dstbuf.at[pl.ds((k - 1) * CH, CH)]],
                            ssems[bn]).wait()

                    pltpu.async_copy(y_sh.at[srcbuf.at[pl.ds((k + 2) * CH, CH)]],
                                     rbufs[bn], gsems[bn])
                else:
                    pltpu.make_async_copy(
                        rbufs[bn],
                        z_sh.at[dstbuf.at[pl.ds((k - 1) * CH, CH)]],
                        ssems[bn]).wait()

                    @pl.when(i < SEG // 3 - 1)
                    def _():
                        pltpu.async_copy(
                            y_sh.at[srcbuf.at[pl.ds((k + 2) * CH, CH)]],
                            rbufs[bn], gsems[bn])

                pltpu.make_async_copy(y_sh.at[srcbuf.at[pl.ds(k * CH, CH)]],
                                      rbufs[b], gsems[b]).wait()
                pltpu.async_copy(rbufs[b],
                                 z_sh.at[dstbuf.at[pl.ds(k * CH, CH)]],
                                 ssems[b], add=True)
            return 0

        lax.fori_loop(0, SEG // 3, body, 0)
        pltpu.make_async_copy(
            rbufs[(SEG - 1) % 3],
            z_sh.at[dstbuf.at[pl.ds((SEG - 1) * CH, CH)]],
            ssems[(SEG - 1) % 3]).wait()
        if h == 1:
            tb = rbufs[0].at[pl.ds(0, TAIL)]
            pltpu.async_copy(y_sh.at[srcbuf.at[pl.ds(SEGE, TAIL)]],
                             tb, gsems[0]).wait()
            pltpu.async_copy(tb, z_sh.at[dstbuf.at[pl.ds(SEGE, TAIL)]],
                             ssems[0], add=True).wait()

    plsc.subcore_barrier()
    for t in range(5):
        off = s * YPT + t * YC
        pltpu.sync_copy(z_sh.at[pl.ds(off, YC)], zb)
        pltpu.sync_copy(zb, z_out.at[c, pl.ds(off, YC)])


@functools.cache
def _build_sc_kernels():
    mesh = plsc.VectorSubcoreMesh(core_axis_name="c", subcore_axis_name="s",
                                  num_cores=2, num_subcores=16)
    params = pltpu.CompilerParams(use_tc_tiling_on_sc=False)
    deg_kernel = pl.kernel(
        _deg_body,
        out_type=jax.ShapeDtypeStruct((2, NP, L), jnp.float32),
        mesh=mesh,
        compiler_params=params,
        scratch_types=[
            pltpu.VMEM_SHARED((NP, L), jnp.float32),
            pltpu.VMEM((EPC,), jnp.int32),
            pltpu.VMEM((CH, L), jnp.float32),
            pltpu.SemaphoreType.DMA,
            pltpu.SemaphoreType.DMA,
        ],
    )
    edge_kernel = pl.kernel(
        _edge_body,
        out_type=jax.ShapeDtypeStruct((2, N, HD), jnp.float32),
        mesh=mesh,
        compiler_params=params,
        scratch_types=[
            pltpu.VMEM_SHARED((N, HD), jnp.float32),
            pltpu.VMEM_SHARED((N, HD), jnp.float32),
            pltpu.VMEM((SEGE + TAIL,), jnp.int32),
            pltpu.VMEM((SEGE + TAIL,), jnp.int32),
            pltpu.VMEM((CH, HD), jnp.float32),
            pltpu.VMEM((CH, HD), jnp.float32),
            pltpu.VMEM((CH, HD), jnp.float32),
            pltpu.SemaphoreType.DMA,
            pltpu.SemaphoreType.DMA,
            pltpu.SemaphoreType.DMA,
            pltpu.SemaphoreType.DMA,
            pltpu.SemaphoreType.DMA,
            pltpu.SemaphoreType.DMA,
        ],
    )
    return deg_kernel, edge_kernel


def _tc_mm_body(x_ref, h_ref, w_ref, xw_ref):
    comb = x_ref[...] + h_ref[...]
    xw_ref[...] = lax.dot_general(comb, w_ref[...], (((1,), (0,)), ((), ())),
                                  precision=lax.Precision.HIGHEST,
                                  preferred_element_type=jnp.float32)


def _tc_scale_body(xw_ref, d0_ref, d1_ref, b_ref, y2_ref, base_ref):
    xw = xw_ref[...]
    deg = d0_ref[0, :, 0:1] + d1_ref[0, :, 0:1] + 1.0
    dis = lax.rsqrt(deg)
    y = xw * dis
    y2_ref[0] = y[:, :HD]
    y2_ref[1] = y[:, HD:]
    base_ref[...] = xw * (dis * dis) + b_ref[...]


def _tc_fin_body(z0_ref, z1_ref, d0_ref, d1_ref, base_ref, o_ref):
    deg = d0_ref[0, :, 0:1] + d1_ref[0, :, 0:1] + 1.0
    dis = lax.rsqrt(deg)
    z = jnp.concatenate([z0_ref[0], z1_ref[0]], axis=1)
    o_ref[...] = z * dis + base_ref[...]


def kernel(x, edge_index, h_cur, c_cur, W, b):
    eix = edge_index.astype(jnp.int32)
    W128 = W[:, :D]
    b128 = b[:D].reshape(1, D)

    deg_kernel, edge_kernel = _build_sc_kernels()
    ones16 = jnp.ones((CH, L), jnp.float32)
    zeros16 = jnp.zeros((RPT, L), jnp.float32)
    zrows = jnp.zeros((YC, HD), jnp.float32)
    deg2 = deg_kernel(eix, ones16, zeros16)

    row_spec = pl.BlockSpec((TCB, D), lambda i: (i, 0))
    half2_spec = pl.BlockSpec((2, TCB, HD), lambda i: (0, i, 0))
    deg0_spec = pl.BlockSpec((1, TCB, L), lambda i: (0, i, 0))
    deg1_spec = pl.BlockSpec((1, TCB, L), lambda i: (1, i, 0))
    xw = pl.pallas_call(
        _tc_mm_body,
        grid=(N // TCB,),
        in_specs=[row_spec, row_spec, pl.BlockSpec((D, D), lambda i: (0, 0))],
        out_specs=row_spec,
        out_shape=jax.ShapeDtypeStruct((N, D), jnp.float32),
    )(x, h_cur, W128)

    y2, base = pl.pallas_call(
        _tc_scale_body,
        grid=(N // TCB,),
        in_specs=[
            row_spec,
            deg0_spec,
            deg1_spec,
            pl.BlockSpec((1, D), lambda i: (0, 0)),
        ],
        out_specs=[half2_spec, row_spec],
        out_shape=[jax.ShapeDtypeStruct((2, N, HD), jnp.float32),
                   jax.ShapeDtypeStruct((N, D), jnp.float32)],
    )(xw, deg2, deg2, b128)

    z2 = edge_kernel(eix, y2, zrows)

    z0_spec = pl.BlockSpec((1, TCB, HD), lambda i: (0, i, 0))
    z1_spec = pl.BlockSpec((1, TCB, HD), lambda i: (1, i, 0))
    out = pl.pallas_call(
        _tc_fin_body,
        grid=(N // TCB,),
        in_specs=[z0_spec, z1_spec, deg0_spec, deg1_spec, row_spec],
        out_specs=row_spec,
        out_shape=jax.ShapeDtypeStruct((N, D), jnp.float32),
    )(z2, z2, deg2, deg2, base)
    return out
